# Initial kernel scaffold; baseline (speedup 1.0000x reference)
import jax
import jax.numpy as jnp
from jax.experimental import pallas as pl


def kernel(x, edge_index, Wl0, Wr0, b0, Wl1, Wr1, b1, Wl2, Wr2, b2):
    raise NotImplementedError("write your pallas kernel here")


# speedup vs baseline: 2.6873x; 2.6873x over previous
import functools

import jax
import jax.numpy as jnp
from jax import lax
from jax.experimental import pallas as pl
from jax.experimental.pallas import tpu as pltpu
from jax.experimental.pallas import tpu_sc as plsc

N = 10000
D = 128
NC = 2
NS = 16
NW = NC * NS
CH = 128
NCHUNK = 80
EPAD = NW * NCHUNK * CH
NP = 10112
RPT = NP // NS


def _sc_agg_body(h_hbm, src_h, dst_h, zrow, parts,
                 sidx, didx, rows, acc, sem):
    c = lax.axis_index("c")
    s = lax.axis_index("s")
    wid = s * NC + c

    @pl.when(s == 0)
    def _init():
        pltpu.sync_copy(zrow, acc)

    plsc.subcore_barrier()

    base = wid * NCHUNK * CH

    def chunk(j, carry):
        off = base + j * CH
        pltpu.sync_copy(src_h.at[pl.ds(off, CH)], sidx)
        pltpu.sync_copy(dst_h.at[pl.ds(off, CH)], didx)
        pltpu.async_copy(h_hbm.at[sidx], rows, sem).wait()
        pltpu.sync_copy(rows, acc.at[didx], add=True)
        return carry

    lax.fori_loop(0, NCHUNK, chunk, 0)
    plsc.subcore_barrier()

    out0 = c * NP + s * RPT
    pltpu.sync_copy(acc.at[pl.ds(s * RPT, RPT)], parts.at[pl.ds(out0, RPT)])


_sc_agg = pl.kernel(
    _sc_agg_body,
    out_type=jax.ShapeDtypeStruct((NC * NP, D), jnp.float32),
    mesh=plsc.VectorSubcoreMesh(core_axis_name="c", subcore_axis_name="s"),
    scratch_types=(
        pltpu.VMEM((CH,), jnp.int32),
        pltpu.VMEM((CH,), jnp.int32),
        pltpu.VMEM((CH, D), jnp.float32),
        pltpu.VMEM_SHARED((NP, D), jnp.float32),
        pltpu.SemaphoreType.DMA,
    ),
)


def _sc_cnt_body(dst_h, zrow, ones_h, cntp, didx, ones_v, cacc):
    c = lax.axis_index("c")
    s = lax.axis_index("s")
    wid = s * NC + c

    @pl.when(s == 0)
    def _init():
        pltpu.sync_copy(zrow, cacc)

    pltpu.sync_copy(ones_h, ones_v)
    plsc.subcore_barrier()

    base = wid * NCHUNK * CH

    def chunk(j, carry):
        off = base + j * CH
        pltpu.sync_copy(dst_h.at[pl.ds(off, CH)], didx)
        pltpu.sync_copy(ones_v, cacc.at[didx], add=True)
        return carry

    lax.fori_loop(0, NCHUNK, chunk, 0)
    plsc.subcore_barrier()

    out0 = c * NP + s * RPT
    pltpu.sync_copy(cacc.at[pl.ds(s * RPT, RPT)], cntp.at[pl.ds(out0, RPT)])


_sc_cnt = pl.kernel(
    _sc_cnt_body,
    out_type=jax.ShapeDtypeStruct((NC * NP, D), jnp.float32),
    mesh=plsc.VectorSubcoreMesh(core_axis_name="c", subcore_axis_name="s"),
    scratch_types=(
        pltpu.VMEM((CH,), jnp.int32),
        pltpu.VMEM((CH, D), jnp.float32),
        pltpu.VMEM_SHARED((NP, D), jnp.float32),
    ),
)


def _tc_layer_body(relu, parts, cntp, h, wl, wr, b, o):
    agg = parts[0] + parts[1]
    cnt = cntp[0, :, 0:1] + cntp[1, :, 0:1]
    mean = agg / jnp.maximum(cnt, 1.0)
    out = (jnp.dot(mean, wl[...], preferred_element_type=jnp.float32)
           + jnp.dot(h[...], wr[...], preferred_element_type=jnp.float32)
           + b[...])
    o[...] = jnp.maximum(out, 0.0) if relu else out


_TCB = 1000


def _make_tc_layer(relu):
    return pl.pallas_call(
        functools.partial(_tc_layer_body, relu),
        grid=(N // _TCB,),
        in_specs=[
            pl.BlockSpec((NC, _TCB, D), lambda j: (0, j, 0)),
            pl.BlockSpec((NC, _TCB, D), lambda j: (0, j, 0)),
            pl.BlockSpec((_TCB, D), lambda j: (j, 0)),
            pl.BlockSpec((D, D), lambda j: (0, 0)),
            pl.BlockSpec((D, D), lambda j: (0, 0)),
            pl.BlockSpec((1, D), lambda j: (0, 0)),
        ],
        out_specs=pl.BlockSpec((_TCB, D), lambda j: (j, 0)),
        out_shape=jax.ShapeDtypeStruct((N, D), jnp.float32),
    )


_tc_layer_relu = _make_tc_layer(True)
_tc_layer_last = _make_tc_layer(False)


def kernel(x, edge_index, Wl0, Wr0, b0, Wl1, Wr1, b1, Wl2, Wr2, b2):
    E = edge_index.shape[1]
    pad = EPAD - E
    src = jnp.concatenate([edge_index[0], jnp.zeros((pad,), jnp.int32)])
    dst = jnp.concatenate([edge_index[1], jnp.full((pad,), N, jnp.int32)])
    zrow = jnp.zeros((NP, D), jnp.float32)
    ones = jnp.ones((CH, D), jnp.float32)

    b0r = b0.reshape(1, D)
    b1r = b1.reshape(1, D)
    b2r = b2.reshape(1, D)

    cntp = _sc_cnt(dst, zrow, ones).reshape(NC, NP, D)
    parts0 = _sc_agg(x, src, dst, zrow).reshape(NC, NP, D)
    h1 = _tc_layer_relu(parts0, cntp, x, Wl0, Wr0, b0r)
    parts1 = _sc_agg(h1, src, dst, zrow).reshape(NC, NP, D)
    h2 = _tc_layer_relu(parts1, cntp, h1, Wl1, Wr1, b1r)
    parts2 = _sc_agg(h2, src, dst, zrow).reshape(NC, NP, D)
    return _tc_layer_last(parts2, cntp, h2, Wl2, Wr2, b2r)


# speedup vs baseline: 2.9535x; 1.0990x over previous
import functools

import jax
import jax.numpy as jnp
from jax import lax
from jax.experimental import pallas as pl
from jax.experimental.pallas import tpu as pltpu
from jax.experimental.pallas import tpu_sc as plsc

N = 10000
D = 128
NC = 2
NS = 16
NW = NC * NS
CH = 256
NCHUNK = 40
EPAD = NW * NCHUNK * CH
NP = 10112
RPT = NP // NS


def _sc_agg_body(h_hbm, src_h, dst_h, zrow, parts,
                 sidx, didx, rows, acc, sem):
    c = lax.axis_index("c")
    s = lax.axis_index("s")
    wid = s * NC + c

    @pl.when(s == 0)
    def _init():
        pltpu.sync_copy(zrow, acc)

    plsc.subcore_barrier()

    base = wid * NCHUNK * CH

    def chunk(j, carry):
        off = base + j * CH
        pltpu.sync_copy(src_h.at[pl.ds(off, CH)], sidx)
        pltpu.sync_copy(dst_h.at[pl.ds(off, CH)], didx)
        pltpu.async_copy(h_hbm.at[sidx], rows, sem).wait()
        pltpu.sync_copy(rows, acc.at[didx], add=True)
        return carry

    lax.fori_loop(0, NCHUNK, chunk, 0)
    plsc.subcore_barrier()

    out0 = c * NP + s * RPT
    pltpu.sync_copy(acc.at[pl.ds(s * RPT, RPT)], parts.at[pl.ds(out0, RPT)])


_sc_agg = pl.kernel(
    _sc_agg_body,
    out_type=jax.ShapeDtypeStruct((NC * NP, D), jnp.float32),
    mesh=plsc.VectorSubcoreMesh(core_axis_name="c", subcore_axis_name="s"),
    scratch_types=(
        pltpu.VMEM((CH,), jnp.int32),
        pltpu.VMEM((CH,), jnp.int32),
        pltpu.VMEM((CH, D), jnp.float32),
        pltpu.VMEM_SHARED((NP, D), jnp.float32),
        pltpu.SemaphoreType.DMA,
    ),
)


def _sc_cnt_body(dst_h, zrow, ones_h, cntp, didx, ones_v, cacc):
    c = lax.axis_index("c")
    s = lax.axis_index("s")
    wid = s * NC + c

    @pl.when(s == 0)
    def _init():
        pltpu.sync_copy(zrow, cacc)

    pltpu.sync_copy(ones_h, ones_v)
    plsc.subcore_barrier()

    base = wid * NCHUNK * CH

    def chunk(j, carry):
        off = base + j * CH
        pltpu.sync_copy(dst_h.at[pl.ds(off, CH)], didx)
        pltpu.sync_copy(ones_v, cacc.at[didx], add=True)
        return carry

    lax.fori_loop(0, NCHUNK, chunk, 0)
    plsc.subcore_barrier()

    out0 = c * NP + s * RPT
    pltpu.sync_copy(cacc.at[pl.ds(s * RPT, RPT)], cntp.at[pl.ds(out0, RPT)])


_sc_cnt = pl.kernel(
    _sc_cnt_body,
    out_type=jax.ShapeDtypeStruct((NC * NP, D), jnp.float32),
    mesh=plsc.VectorSubcoreMesh(core_axis_name="c", subcore_axis_name="s"),
    scratch_types=(
        pltpu.VMEM((CH,), jnp.int32),
        pltpu.VMEM((CH, D), jnp.float32),
        pltpu.VMEM_SHARED((NP, D), jnp.float32),
    ),
)


def _tc_layer_body(relu, parts, cntp, h, wl, wr, b, o):
    agg = parts[0] + parts[1]
    cnt = cntp[0, :, 0:1] + cntp[1, :, 0:1]
    mean = agg / jnp.maximum(cnt, 1.0)
    out = (jnp.dot(mean, wl[...], preferred_element_type=jnp.float32)
           + jnp.dot(h[...], wr[...], preferred_element_type=jnp.float32)
           + b[...])
    o[...] = jnp.maximum(out, 0.0) if relu else out


_TCB = 1000


def _make_tc_layer(relu):
    return pl.pallas_call(
        functools.partial(_tc_layer_body, relu),
        grid=(N // _TCB,),
        in_specs=[
            pl.BlockSpec((NC, _TCB, D), lambda j: (0, j, 0)),
            pl.BlockSpec((NC, _TCB, D), lambda j: (0, j, 0)),
            pl.BlockSpec((_TCB, D), lambda j: (j, 0)),
            pl.BlockSpec((D, D), lambda j: (0, 0)),
            pl.BlockSpec((D, D), lambda j: (0, 0)),
            pl.BlockSpec((1, D), lambda j: (0, 0)),
        ],
        out_specs=pl.BlockSpec((_TCB, D), lambda j: (j, 0)),
        out_shape=jax.ShapeDtypeStruct((N, D), jnp.float32),
    )


_tc_layer_relu = _make_tc_layer(True)
_tc_layer_last = _make_tc_layer(False)


def kernel(x, edge_index, Wl0, Wr0, b0, Wl1, Wr1, b1, Wl2, Wr2, b2):
    E = edge_index.shape[1]
    pad = EPAD - E
    src = jnp.concatenate([edge_index[0], jnp.zeros((pad,), jnp.int32)])
    dst = jnp.concatenate([edge_index[1], jnp.full((pad,), N, jnp.int32)])
    zrow = jnp.zeros((NP, D), jnp.float32)
    ones = jnp.ones((CH, D), jnp.float32)

    b0r = b0.reshape(1, D)
    b1r = b1.reshape(1, D)
    b2r = b2.reshape(1, D)

    cntp = _sc_cnt(dst, zrow, ones).reshape(NC, NP, D)
    parts0 = _sc_agg(x, src, dst, zrow).reshape(NC, NP, D)
    h1 = _tc_layer_relu(parts0, cntp, x, Wl0, Wr0, b0r)
    parts1 = _sc_agg(h1, src, dst, zrow).reshape(NC, NP, D)
    h2 = _tc_layer_relu(parts1, cntp, h1, Wl1, Wr1, b1r)
    parts2 = _sc_agg(h2, src, dst, zrow).reshape(NC, NP, D)
    return _tc_layer_last(parts2, cntp, h2, Wl2, Wr2, b2r)


# speedup vs baseline: 3.0712x; 1.0399x over previous
import functools

import jax
import jax.numpy as jnp
from jax import lax
from jax.experimental import pallas as pl
from jax.experimental.pallas import tpu as pltpu
from jax.experimental.pallas import tpu_sc as plsc

N = 10000
D = 128
NC = 2
NS = 16
NW = NC * NS
CH = 256
NCHUNK = 40
EPAD = NW * NCHUNK * CH
NP = 10112
RPT = NP // NS


def _sc_agg_body(h_hbm, src_h, dst_h, zrow, parts,
                 sidx0, sidx1, didx0, didx1, rows, acc,
                 semS0, semS1, semD0, semD1, semG):
    c = lax.axis_index("c")
    s = lax.axis_index("s")
    wid = s * NC + c

    @pl.when(s == 0)
    def _init():
        pltpu.sync_copy(zrow, acc)

    plsc.subcore_barrier()

    base = wid * NCHUNK * CH
    sbuf = (sidx0, sidx1)
    dbuf = (didx0, didx1)
    ssem = (semS0, semS1)
    dsem = (semD0, semD1)
    hs = [None, None]
    hd = [None, None]

    hs[0] = pltpu.async_copy(src_h.at[pl.ds(base, CH)], sbuf[0], ssem[0])
    hd[0] = pltpu.async_copy(dst_h.at[pl.ds(base, CH)], dbuf[0], dsem[0])
    for g in range(NCHUNK):
        p = g % 2
        q = (g + 1) % 2
        if g + 1 < NCHUNK:
            off = base + (g + 1) * CH
            hs[q] = pltpu.async_copy(src_h.at[pl.ds(off, CH)], sbuf[q], ssem[q])
            hd[q] = pltpu.async_copy(dst_h.at[pl.ds(off, CH)], dbuf[q], dsem[q])
        hs[p].wait()
        pltpu.async_copy(h_hbm.at[sbuf[p]], rows, semG).wait()
        hd[p].wait()
        pltpu.sync_copy(rows, acc.at[dbuf[p]], add=True)

    plsc.subcore_barrier()

    out0 = c * NP + s * RPT
    pltpu.sync_copy(acc.at[pl.ds(s * RPT, RPT)], parts.at[pl.ds(out0, RPT)])


_sc_agg = pl.kernel(
    _sc_agg_body,
    out_type=jax.ShapeDtypeStruct((NC * NP, D), jnp.float32),
    mesh=plsc.VectorSubcoreMesh(core_axis_name="c", subcore_axis_name="s"),
    scratch_types=(
        pltpu.VMEM((CH,), jnp.int32),
        pltpu.VMEM((CH,), jnp.int32),
        pltpu.VMEM((CH,), jnp.int32),
        pltpu.VMEM((CH,), jnp.int32),
        pltpu.VMEM((CH, D), jnp.float32),
        pltpu.VMEM_SHARED((NP, D), jnp.float32),
        pltpu.SemaphoreType.DMA,
        pltpu.SemaphoreType.DMA,
        pltpu.SemaphoreType.DMA,
        pltpu.SemaphoreType.DMA,
        pltpu.SemaphoreType.DMA,
    ),
)


def _sc_cnt_body(dst_h, zrow, ones_h, cntp, didx, ones_v, cacc):
    c = lax.axis_index("c")
    s = lax.axis_index("s")
    wid = s * NC + c

    @pl.when(s == 0)
    def _init():
        pltpu.sync_copy(zrow, cacc)

    pltpu.sync_copy(ones_h, ones_v)
    plsc.subcore_barrier()

    base = wid * NCHUNK * CH

    def chunk(j, carry):
        off = base + j * CH
        pltpu.sync_copy(dst_h.at[pl.ds(off, CH)], didx)
        pltpu.sync_copy(ones_v, cacc.at[didx], add=True)
        return carry

    lax.fori_loop(0, NCHUNK, chunk, 0)
    plsc.subcore_barrier()

    out0 = c * NP + s * RPT
    pltpu.sync_copy(cacc.at[pl.ds(s * RPT, RPT)], cntp.at[pl.ds(out0, RPT)])


_sc_cnt = pl.kernel(
    _sc_cnt_body,
    out_type=jax.ShapeDtypeStruct((NC * NP, D), jnp.float32),
    mesh=plsc.VectorSubcoreMesh(core_axis_name="c", subcore_axis_name="s"),
    scratch_types=(
        pltpu.VMEM((CH,), jnp.int32),
        pltpu.VMEM((CH, D), jnp.float32),
        pltpu.VMEM_SHARED((NP, D), jnp.float32),
    ),
)


def _tc_layer_body(relu, parts, cntp, h, wl, wr, b, o):
    agg = parts[0] + parts[1]
    cnt = cntp[0, :, 0:1] + cntp[1, :, 0:1]
    mean = agg / jnp.maximum(cnt, 1.0)
    out = (jnp.dot(mean, wl[...], preferred_element_type=jnp.float32)
           + jnp.dot(h[...], wr[...], preferred_element_type=jnp.float32)
           + b[...])
    o[...] = jnp.maximum(out, 0.0) if relu else out


_TCB = 1000


def _make_tc_layer(relu):
    return pl.pallas_call(
        functools.partial(_tc_layer_body, relu),
        grid=(N // _TCB,),
        in_specs=[
            pl.BlockSpec((NC, _TCB, D), lambda j: (0, j, 0)),
            pl.BlockSpec((NC, _TCB, D), lambda j: (0, j, 0)),
            pl.BlockSpec((_TCB, D), lambda j: (j, 0)),
            pl.BlockSpec((D, D), lambda j: (0, 0)),
            pl.BlockSpec((D, D), lambda j: (0, 0)),
            pl.BlockSpec((1, D), lambda j: (0, 0)),
        ],
        out_specs=pl.BlockSpec((_TCB, D), lambda j: (j, 0)),
        out_shape=jax.ShapeDtypeStruct((N, D), jnp.float32),
    )


_tc_layer_relu = _make_tc_layer(True)
_tc_layer_last = _make_tc_layer(False)


def kernel(x, edge_index, Wl0, Wr0, b0, Wl1, Wr1, b1, Wl2, Wr2, b2):
    E = edge_index.shape[1]
    pad = EPAD - E
    src = jnp.concatenate([edge_index[0], jnp.zeros((pad,), jnp.int32)])
    dst = jnp.concatenate([edge_index[1], jnp.full((pad,), N, jnp.int32)])
    zrow = jnp.zeros((NP, D), jnp.float32)
    ones = jnp.ones((CH, D), jnp.float32)

    b0r = b0.reshape(1, D)
    b1r = b1.reshape(1, D)
    b2r = b2.reshape(1, D)

    cntp = _sc_cnt(dst, zrow, ones).reshape(NC, NP, D)
    parts0 = _sc_agg(x, src, dst, zrow).reshape(NC, NP, D)
    h1 = _tc_layer_relu(parts0, cntp, x, Wl0, Wr0, b0r)
    parts1 = _sc_agg(h1, src, dst, zrow).reshape(NC, NP, D)
    h2 = _tc_layer_relu(parts1, cntp, h1, Wl1, Wr1, b1r)
    parts2 = _sc_agg(h2, src, dst, zrow).reshape(NC, NP, D)
    return _tc_layer_last(parts2, cntp, h2, Wl2, Wr2, b2r)
